# baseline (speedup 1.0000x reference)
import functools

import jax
import jax.numpy as jnp
from jax import lax
from jax.experimental import pallas as pl
from jax.experimental.pallas import tpu as pltpu
from jax.experimental.pallas import tpu_sc as plsc

VOCAB = 100000
HID = 768
MAX_POS = 4096
B, S = 4, 4096
TOK = B * S
EPS = 1e-12

NC, NS, L = 2, 16, 16
NW = NC * NS
TPW = TOK // NW
C = 32
NCHUNK = TPW // C
NG = C // L

_MESH = plsc.VectorSubcoreMesh(
    core_axis_name="c", subcore_axis_name="s", num_cores=NC, num_subcores=NS)


def _rsqrt(v):
    vi = lax.bitcast_convert_type(v, jnp.int32)
    y = lax.bitcast_convert_type(jnp.int32(0x5F3759DF) - (vi >> 1), jnp.float32)
    for _ in range(3):
        y = y * (1.5 - 0.5 * v * y * y)
    return y


@functools.partial(
    pl.kernel,
    out_type=jax.ShapeDtypeStruct((TOK, HID), jnp.float32),
    mesh=_MESH,
    scratch_types=dict(
        idx_v=pltpu.VMEM((C,), jnp.int32),
        pid_v=pltpu.VMEM((C,), jnp.int32),
        ttv=pltpu.VMEM((C,), jnp.int32),
        wrows=pltpu.VMEM((C, HID), jnp.float32),
        prows=pltpu.VMEM((C, HID), jnp.float32),
        ttab_v=pltpu.VMEM((2, HID), jnp.float32),
        gam_v=pltpu.VMEM((HID,), jnp.float32),
        bet_v=pltpu.VMEM((HID,), jnp.float32),
        xbuf=pltpu.VMEM((HID, L), jnp.float32),
        sem_w=pltpu.SemaphoreType.DMA,
        sem_p=pltpu.SemaphoreType.DMA,
    ),
    compiler_params=pltpu.CompilerParams(
        use_tc_tiling_on_sc=False, needs_layout_passes=False),
)
def _sc_embed(ids_hbm, tt_hbm, pos_hbm, word_hbm, ttab_hbm, pemb_hbm,
              gam_hbm, bet_hbm, out_hbm, *, idx_v, pid_v, ttv, wrows, prows,
              ttab_v, gam_v, bet_v, xbuf, sem_w, sem_p):
    wid = lax.axis_index("s") * NC + lax.axis_index("c")
    base = wid * TPW

    pltpu.sync_copy(ttab_hbm, ttab_v)
    pltpu.sync_copy(gam_hbm, gam_v)
    pltpu.sync_copy(bet_hbm, bet_v)

    lanes = lax.iota(jnp.int32, L)

    def chunk_body(ck, carry):
        tok0 = base + ck * C
        pltpu.sync_copy(ids_hbm.at[pl.ds(tok0, C)], idx_v)
        pltpu.sync_copy(pos_hbm.at[pl.ds(tok0, C)], pid_v)
        pltpu.sync_copy(tt_hbm.at[pl.ds(tok0, C)], ttv)
        cw = pltpu.make_async_copy(word_hbm.at[idx_v], wrows, sem_w)
        cp = pltpu.make_async_copy(pemb_hbm.at[pid_v], prows, sem_p)
        cw.start()
        cp.start()
        cw.wait()
        cp.wait()

        for g in range(NG):
            rowi = g * L + lanes
            ttg = ttv[pl.ds(g * L, L)]

            def p1(h, sq):
                s, q = sq
                hv = jnp.full((L,), h, jnp.int32)
                x = (plsc.load_gather(wrows, [rowi, hv])
                     + plsc.load_gather(prows, [rowi, hv])
                     + plsc.load_gather(ttab_v, [ttg, hv]))
                xbuf[h, :] = x
                return s + x, q + x * x

            zero = jnp.zeros((L,), jnp.float32)
            ssum, qsum = lax.fori_loop(0, HID, p1, (zero, zero))
            mean = ssum * (1.0 / HID)
            var = qsum * (1.0 / HID) - mean * mean
            rstd = _rsqrt(var + EPS)

            def p2(h, _):
                hv = jnp.full((L,), h, jnp.int32)
                x = xbuf[h, :]
                gv = plsc.load_gather(gam_v, [hv])
                bv = plsc.load_gather(bet_v, [hv])
                y = (x - mean) * rstd * gv + bv
                plsc.store_scatter(wrows, [rowi, hv], y)
                return 0

            lax.fori_loop(0, HID, p2, 0)

        pltpu.sync_copy(wrows, out_hbm.at[pl.ds(tok0, C)])
        return carry

    lax.fori_loop(0, NCHUNK, chunk_body, 0)


@jax.jit
def kernel(input_ids, token_type_ids, position_ids, word_emb, token_type_emb,
           position_emb, ln_gamma, ln_beta):
    ids = input_ids.reshape(-1).astype(jnp.int32)
    tts = token_type_ids.reshape(-1).astype(jnp.int32)
    pos = position_ids.reshape(-1).astype(jnp.int32)
    out = _sc_embed(ids, tts, pos, word_emb, token_type_emb, position_emb,
                    ln_gamma, ln_beta)
    return out.reshape(B, S, HID)


# speedup vs baseline: 1.2696x; 1.2696x over previous
import functools

import jax
import jax.numpy as jnp
from jax import lax
from jax.experimental import pallas as pl
from jax.experimental.pallas import tpu as pltpu
from jax.experimental.pallas import tpu_sc as plsc

VOCAB = 100000
HID = 768
MAX_POS = 4096
B, S = 4, 4096
TOK = B * S
EPS = 1e-12

NC, NS, L = 2, 16, 16
NW = NC * NS
TPW = TOK // NW
C = L
NCHUNK = TPW // C
U = 16

_MESH = plsc.VectorSubcoreMesh(
    core_axis_name="c", subcore_axis_name="s", num_cores=NC, num_subcores=NS)


def _rsqrt(v):
    vi = lax.bitcast_convert_type(v, jnp.int32)
    y = lax.bitcast_convert_type(jnp.int32(0x5F3759DF) - (vi >> 1), jnp.float32)
    for _ in range(3):
        y = y * (1.5 - 0.5 * v * y * y)
    return y


@functools.partial(
    pl.kernel,
    out_type=jax.ShapeDtypeStruct((TOK, HID), jnp.float32),
    mesh=_MESH,
    scratch_types=dict(
        ids2=pltpu.VMEM((NCHUNK, C), jnp.int32),
        pos2=pltpu.VMEM((NCHUNK, C), jnp.int32),
        tt2=pltpu.VMEM((NCHUNK, C), jnp.int32),
        wrows=pltpu.VMEM((2 * C, HID), jnp.float32),
        prows=pltpu.VMEM((2 * C, HID), jnp.float32),
        ybuf=pltpu.VMEM((2 * C, HID), jnp.float32),
        ttab_v=pltpu.VMEM((2, HID), jnp.float32),
        gam_v=pltpu.VMEM((HID,), jnp.float32),
        bet_v=pltpu.VMEM((HID,), jnp.float32),
        xbuf=pltpu.VMEM((HID, L), jnp.float32),
        sem_w0=pltpu.SemaphoreType.DMA,
        sem_p0=pltpu.SemaphoreType.DMA,
        sem_o0=pltpu.SemaphoreType.DMA,
        sem_w1=pltpu.SemaphoreType.DMA,
        sem_p1=pltpu.SemaphoreType.DMA,
        sem_o1=pltpu.SemaphoreType.DMA,
    ),
    compiler_params=pltpu.CompilerParams(
        use_tc_tiling_on_sc=False, needs_layout_passes=False),
)
def _sc_embed(ids_hbm, tt_hbm, pos_hbm, word_hbm, ttab_hbm, pemb_hbm,
              gam_hbm, bet_hbm, out_hbm, *, ids2, pos2, tt2, wrows, prows,
              ybuf, ttab_v, gam_v, bet_v, xbuf,
              sem_w0, sem_p0, sem_o0, sem_w1, sem_p1, sem_o1):
    wid = lax.axis_index("s") * NC + lax.axis_index("c")
    base = wid * TPW

    pltpu.sync_copy(ids_hbm.at[pl.ds(wid * NCHUNK, NCHUNK)], ids2)
    pltpu.sync_copy(pos_hbm.at[pl.ds(wid * NCHUNK, NCHUNK)], pos2)
    pltpu.sync_copy(tt_hbm.at[pl.ds(wid * NCHUNK, NCHUNK)], tt2)
    pltpu.sync_copy(ttab_hbm, ttab_v)
    pltpu.sync_copy(gam_hbm, gam_v)
    pltpu.sync_copy(bet_hbm, bet_v)

    lanes = lax.iota(jnp.int32, L)
    sems = ((sem_w0, sem_p0, sem_o0), (sem_w1, sem_p1, sem_o1))

    def start_gather(ck, par):
        sw, sp, _ = sems[par]
        dst_w = wrows.at[pl.ds(par * C, C)]
        dst_p = prows.at[pl.ds(par * C, C)]
        pltpu.make_async_copy(word_hbm.at[ids2.at[ck]], dst_w, sw).start()
        pltpu.make_async_copy(pemb_hbm.at[pos2.at[ck]], dst_p, sp).start()

    def wait_gather(ck, par):
        sw, sp, _ = sems[par]
        pltpu.make_async_copy(word_hbm.at[ids2.at[ck]],
                              wrows.at[pl.ds(par * C, C)], sw).wait()
        pltpu.make_async_copy(pemb_hbm.at[pos2.at[ck]],
                              prows.at[pl.ds(par * C, C)], sp).wait()

    def out_copy(ck, par):
        _, _, so = sems[par]
        return pltpu.make_async_copy(
            ybuf.at[pl.ds(par * C, C)],
            out_hbm.at[pl.ds(base + ck * C, C)], so)

    def compute(ck, par):
        rowg = par * C + lanes
        ttg = tt2[ck, :]
        ttf = jnp.where(ttg == 1, 1.0, 0.0).astype(jnp.float32)

        def p1(hb, sq):
            s, q = sq
            h0 = hb * U
            tt0v = ttab_v[0, pl.ds(h0, U)]
            dttv = ttab_v[1, pl.ds(h0, U)] - tt0v
            for u in range(U):
                h = h0 + u
                hv = jnp.full((L,), h, jnp.int32)
                x = (plsc.load_gather(wrows, [rowg, hv])
                     + plsc.load_gather(prows, [rowg, hv])
                     + (ttf * dttv[u] + tt0v[u]))
                xbuf[h, :] = x
                s = s + x
                q = q + x * x
            return s, q

        zero = jnp.zeros((L,), jnp.float32)
        ssum, qsum = lax.fori_loop(0, HID // U, p1, (zero, zero))
        mean = ssum * (1.0 / HID)
        var = qsum * (1.0 / HID) - mean * mean
        rstd = _rsqrt(var + EPS)

        def p2(hb, _):
            h0 = hb * U
            gv = gam_v[pl.ds(h0, U)]
            bv = bet_v[pl.ds(h0, U)]
            for u in range(U):
                h = h0 + u
                hv = jnp.full((L,), h, jnp.int32)
                a = rstd * gv[u]
                b2 = bv[u] - mean * a
                y = xbuf[h, :] * a + b2
                plsc.store_scatter(ybuf, [rowg, hv], y)
            return 0

        lax.fori_loop(0, HID // U, p2, 0)

    start_gather(0, 0)

    def pair_body(i, carry):
        ck0 = 2 * i
        ck1 = ck0 + 1
        start_gather(ck1, 1)
        wait_gather(ck0, 0)

        @pl.when(i > 0)
        def _():
            out_copy(ck0 - 2, 0).wait()

        compute(ck0, 0)
        out_copy(ck0, 0).start()

        @pl.when(i < NCHUNK // 2 - 1)
        def _():
            start_gather(ck0 + 2, 0)

        wait_gather(ck1, 1)

        @pl.when(i > 0)
        def _():
            out_copy(ck1 - 2, 1).wait()

        compute(ck1, 1)
        out_copy(ck1, 1).start()
        return carry

    lax.fori_loop(0, NCHUNK // 2, pair_body, 0)
    out_copy(NCHUNK - 2, 0).wait()
    out_copy(NCHUNK - 1, 1).wait()


@jax.jit
def kernel(input_ids, token_type_ids, position_ids, word_emb, token_type_emb,
           position_emb, ln_gamma, ln_beta):
    ids = input_ids.reshape(TOK // C, C).astype(jnp.int32)
    tts = token_type_ids.reshape(TOK // C, C).astype(jnp.int32)
    pos = position_ids.reshape(TOK // C, C).astype(jnp.int32)
    out = _sc_embed(ids, tts, pos, word_emb, token_type_emb, position_emb,
                    ln_gamma, ln_beta)
    return out.reshape(B, S, HID)


# speedup vs baseline: 2.7341x; 2.1535x over previous
import functools

import jax
import jax.numpy as jnp
from jax import lax
from jax.experimental import pallas as pl
from jax.experimental.pallas import tpu as pltpu
from jax.experimental.pallas import tpu_sc as plsc

VOCAB = 100000
HID = 768
MAX_POS = 4096
B, S = 4, 4096
TOK = B * S
EPS = 1e-12

NC, NS, L = 2, 16, 16
NW = NC * NS
TPW = TOK // NW
C = L
NCHUNK = TPW // C
U = 16

_MESH = plsc.VectorSubcoreMesh(
    core_axis_name="c", subcore_axis_name="s", num_cores=NC, num_subcores=NS)


def _rsqrt(v):
    vi = lax.bitcast_convert_type(v, jnp.int32)
    y = lax.bitcast_convert_type(jnp.int32(0x5F3759DF) - (vi >> 1), jnp.float32)
    for _ in range(3):
        y = y * (1.5 - 0.5 * v * y * y)
    return y


@functools.partial(
    pl.kernel,
    out_type=jax.ShapeDtypeStruct((TOK, HID), jnp.float32),
    mesh=_MESH,
    scratch_types=dict(
        ids2=pltpu.VMEM((NCHUNK, C), jnp.int32),
        pos2=pltpu.VMEM((NCHUNK, C), jnp.int32),
        tt2=pltpu.VMEM((NCHUNK, C), jnp.int32),
        wrows=pltpu.VMEM((2 * C, HID), jnp.float32),
        prows=pltpu.VMEM((2 * C, HID), jnp.float32),
        ybuf=pltpu.VMEM((2 * C, HID), jnp.float32),
        ttab_v=pltpu.VMEM((2, HID), jnp.float32),
        gam_v=pltpu.VMEM((HID,), jnp.float32),
        bet_v=pltpu.VMEM((HID,), jnp.float32),
        sem_w0=pltpu.SemaphoreType.DMA,
        sem_p0=pltpu.SemaphoreType.DMA,
        sem_o0=pltpu.SemaphoreType.DMA,
        sem_w1=pltpu.SemaphoreType.DMA,
        sem_p1=pltpu.SemaphoreType.DMA,
        sem_o1=pltpu.SemaphoreType.DMA,
    ),
    compiler_params=pltpu.CompilerParams(
        use_tc_tiling_on_sc=False, needs_layout_passes=False),
)
def _sc_embed(ids_hbm, tt_hbm, pos_hbm, word_hbm, ttab_hbm, pemb_hbm,
              gam_hbm, bet_hbm, out_hbm, *, ids2, pos2, tt2, wrows, prows,
              ybuf, ttab_v, gam_v, bet_v,
              sem_w0, sem_p0, sem_o0, sem_w1, sem_p1, sem_o1):
    wid = lax.axis_index("s") * NC + lax.axis_index("c")
    base = wid * TPW

    pltpu.sync_copy(ids_hbm.at[pl.ds(wid * NCHUNK, NCHUNK)], ids2)
    pltpu.sync_copy(pos_hbm.at[pl.ds(wid * NCHUNK, NCHUNK)], pos2)
    pltpu.sync_copy(tt_hbm.at[pl.ds(wid * NCHUNK, NCHUNK)], tt2)
    pltpu.sync_copy(ttab_hbm, ttab_v)
    pltpu.sync_copy(gam_hbm, gam_v)
    pltpu.sync_copy(bet_hbm, bet_v)

    sems = ((sem_w0, sem_p0, sem_o0), (sem_w1, sem_p1, sem_o1))

    def start_gather(ck, par):
        sw, sp, _ = sems[par]
        dst_w = wrows.at[pl.ds(par * C, C)]
        dst_p = prows.at[pl.ds(par * C, C)]
        pltpu.make_async_copy(word_hbm.at[ids2.at[ck]], dst_w, sw).start()
        pltpu.make_async_copy(pemb_hbm.at[pos2.at[ck]], dst_p, sp).start()

    def wait_gather(ck, par):
        sw, sp, _ = sems[par]
        pltpu.make_async_copy(word_hbm.at[ids2.at[ck]],
                              wrows.at[pl.ds(par * C, C)], sw).wait()
        pltpu.make_async_copy(pemb_hbm.at[pos2.at[ck]],
                              prows.at[pl.ds(par * C, C)], sp).wait()

    def out_copy(ck, par):
        _, _, so = sems[par]
        return pltpu.make_async_copy(
            ybuf.at[pl.ds(par * C, C)],
            out_hbm.at[pl.ds(base + ck * C, C)], so)

    inv_h = jnp.full((L,), 1.0 / HID, jnp.float32)
    eps_v = jnp.full((L,), EPS, jnp.float32)

    def compute(ck, par):
        ck_v = jnp.full((L,), ck, jnp.int32)

        def tok_pair(tp, carry):
            r0 = par * C + 2 * tp
            r1 = r0 + 1
            ta = plsc.load_gather(tt2, [ck_v, jnp.full((L,), 0, jnp.int32) + 2 * tp])
            tb = plsc.load_gather(tt2, [ck_v, jnp.full((L,), 1, jnp.int32) + 2 * tp])
            ma = ta == 1
            mb = tb == 1
            z = jnp.zeros((L,), jnp.float32)
            sa0 = sa1 = qa0 = qa1 = z
            sb0 = sb1 = qb0 = qb1 = z
            for j in range(HID // L):
                sl = pl.ds(j * L, L)
                t0v = ttab_v[0, sl]
                t1v = ttab_v[1, sl]
                xa = wrows[r0, sl] + prows[r0, sl] + jnp.where(ma, t1v, t0v)
                xb = wrows[r1, sl] + prows[r1, sl] + jnp.where(mb, t1v, t0v)
                ybuf[r0, sl] = xa
                ybuf[r1, sl] = xb
                if j % 2 == 0:
                    sa0 = sa0 + xa
                    qa0 = qa0 + xa * xa
                    sb0 = sb0 + xb
                    qb0 = qb0 + xb * xb
                else:
                    sa1 = sa1 + xa
                    qa1 = qa1 + xa * xa
                    sb1 = sb1 + xb
                    qb1 = qb1 + xb * xb

            ma_v = jnp.sum(sa0 + sa1) * inv_h
            mb_v = jnp.sum(sb0 + sb1) * inv_h
            va = jnp.sum(qa0 + qa1) * inv_h - ma_v * ma_v + eps_v
            vb = jnp.sum(qb0 + qb1) * inv_h - mb_v * mb_v + eps_v
            ra = _rsqrt(va)
            rb = _rsqrt(vb)
            na = -(ma_v * ra)
            nb = -(mb_v * rb)

            for j in range(HID // L):
                sl = pl.ds(j * L, L)
                g = gam_v[sl]
                b = bet_v[sl]
                ybuf[r0, sl] = (ybuf[r0, sl] * ra + na) * g + b
                ybuf[r1, sl] = (ybuf[r1, sl] * rb + nb) * g + b
            return carry

        lax.fori_loop(0, C // 2, tok_pair, 0)

    start_gather(0, 0)

    def pair_body(i, carry):
        ck0 = 2 * i
        ck1 = ck0 + 1
        start_gather(ck1, 1)
        wait_gather(ck0, 0)

        @pl.when(i > 0)
        def _():
            out_copy(ck0 - 2, 0).wait()

        compute(ck0, 0)
        out_copy(ck0, 0).start()

        @pl.when(i < NCHUNK // 2 - 1)
        def _():
            start_gather(ck0 + 2, 0)

        wait_gather(ck1, 1)

        @pl.when(i > 0)
        def _():
            out_copy(ck1 - 2, 1).wait()

        compute(ck1, 1)
        out_copy(ck1, 1).start()
        return carry

    lax.fori_loop(0, NCHUNK // 2, pair_body, 0)
    out_copy(NCHUNK - 2, 0).wait()
    out_copy(NCHUNK - 1, 1).wait()


@jax.jit
def kernel(input_ids, token_type_ids, position_ids, word_emb, token_type_emb,
           position_emb, ln_gamma, ln_beta):
    ids = input_ids.reshape(TOK // C, C).astype(jnp.int32)
    tts = token_type_ids.reshape(TOK // C, C).astype(jnp.int32)
    pos = position_ids.reshape(TOK // C, C).astype(jnp.int32)
    out = _sc_embed(ids, tts, pos, word_emb, token_type_emb, position_emb,
                    ln_gamma, ln_beta)
    return out.reshape(B, S, HID)


# speedup vs baseline: 4.3609x; 1.5950x over previous
import functools

import jax
import jax.numpy as jnp
from jax import lax
from jax.experimental import pallas as pl
from jax.experimental.pallas import tpu as pltpu
from jax.experimental.pallas import tpu_sc as plsc

VOCAB = 100000
HID = 768
MAX_POS = 4096
B, S = 4, 4096
TOK = B * S
EPS = 1e-12

NC, NS, L = 2, 16, 16
NW = NC * NS
TPW = TOK // NW
C = L
NCHUNK = TPW // C

_MESH = plsc.VectorSubcoreMesh(
    core_axis_name="c", subcore_axis_name="s", num_cores=NC, num_subcores=NS)


def _rsqrt(v):
    vi = lax.bitcast_convert_type(v, jnp.int32)
    y = lax.bitcast_convert_type(jnp.int32(0x5F3759DF) - (vi >> 1), jnp.float32)
    for _ in range(3):
        y = y * (1.5 - 0.5 * v * y * y)
    return y


@functools.partial(
    pl.kernel,
    out_type=jax.ShapeDtypeStruct((B, S, HID), jnp.float32),
    mesh=_MESH,
    scratch_types=dict(
        ids_l=pltpu.VMEM((TPW,), jnp.int32),
        pos_l=pltpu.VMEM((TPW,), jnp.int32),
        tt_l=pltpu.VMEM((TPW,), jnp.int32),
        wrows=pltpu.VMEM((2 * C, HID), jnp.float32),
        prows=pltpu.VMEM((2 * C, HID), jnp.float32),
        ybuf=pltpu.VMEM((2 * C, HID), jnp.float32),
        ttab_v=pltpu.VMEM((2, HID), jnp.float32),
        gam_v=pltpu.VMEM((HID,), jnp.float32),
        bet_v=pltpu.VMEM((HID,), jnp.float32),
        sem_w0=pltpu.SemaphoreType.DMA,
        sem_p0=pltpu.SemaphoreType.DMA,
        sem_o0=pltpu.SemaphoreType.DMA,
        sem_w1=pltpu.SemaphoreType.DMA,
        sem_p1=pltpu.SemaphoreType.DMA,
        sem_o1=pltpu.SemaphoreType.DMA,
    ),
    compiler_params=pltpu.CompilerParams(
        use_tc_tiling_on_sc=True, needs_layout_passes=False),
)
def _sc_embed(ids_hbm, tt_hbm, pos_hbm, word_hbm, ttab_hbm, pemb_hbm,
              gam_hbm, bet_hbm, out_hbm, *, ids_l, pos_l, tt_l, wrows, prows,
              ybuf, ttab_v, gam_v, bet_v,
              sem_w0, sem_p0, sem_o0, sem_w1, sem_p1, sem_o1):
    wid = lax.axis_index("s") * NC + lax.axis_index("c")
    base = wid * TPW

    pltpu.sync_copy(ids_hbm.at[pl.ds(base, TPW)], ids_l)
    pltpu.sync_copy(pos_hbm.at[pl.ds(base, TPW)], pos_l)
    pltpu.sync_copy(tt_hbm.at[pl.ds(base, TPW)], tt_l)
    pltpu.sync_copy(ttab_hbm, ttab_v)
    pltpu.sync_copy(gam_hbm, gam_v)
    pltpu.sync_copy(bet_hbm, bet_v)

    sems = ((sem_w0, sem_p0, sem_o0), (sem_w1, sem_p1, sem_o1))

    def start_gather(ck, par):
        sw, sp, _ = sems[par]
        idx = ids_l.at[pl.ds(ck * C, C)]
        pdx = pos_l.at[pl.ds(ck * C, C)]
        pltpu.make_async_copy(word_hbm.at[idx],
                              wrows.at[pl.ds(par * C, C)], sw).start()
        pltpu.make_async_copy(pemb_hbm.at[pdx],
                              prows.at[pl.ds(par * C, C)], sp).start()

    def wait_gather(ck, par):
        sw, sp, _ = sems[par]
        idx = ids_l.at[pl.ds(ck * C, C)]
        pdx = pos_l.at[pl.ds(ck * C, C)]
        pltpu.make_async_copy(word_hbm.at[idx],
                              wrows.at[pl.ds(par * C, C)], sw).wait()
        pltpu.make_async_copy(pemb_hbm.at[pdx],
                              prows.at[pl.ds(par * C, C)], sp).wait()

    def out_copy(ck, par):
        _, _, so = sems[par]
        tok0 = base + ck * C
        b = tok0 // S
        s0 = tok0 - b * S
        return pltpu.make_async_copy(
            ybuf.at[pl.ds(par * C, C)],
            out_hbm.at[b, pl.ds(s0, C)], so)

    inv_h = jnp.full((L,), 1.0 / HID, jnp.float32)
    eps_v = jnp.full((L,), EPS, jnp.float32)

    def compute(ck, par):
        def tok_pair(tp, carry):
            r0 = par * C + 2 * tp
            r1 = r0 + 1
            t0i = ck * C + 2 * tp
            ta = plsc.load_gather(tt_l, [jnp.full((L,), 0, jnp.int32) + t0i])
            tb = plsc.load_gather(tt_l, [jnp.full((L,), 1, jnp.int32) + t0i])
            ma = ta == 1
            mb = tb == 1
            z = jnp.zeros((L,), jnp.float32)
            sa0 = sa1 = qa0 = qa1 = z
            sb0 = sb1 = qb0 = qb1 = z
            for j in range(HID // L):
                sl = pl.ds(j * L, L)
                t0v = ttab_v[0, sl]
                t1v = ttab_v[1, sl]
                xa = wrows[r0, sl] + prows[r0, sl] + jnp.where(ma, t1v, t0v)
                xb = wrows[r1, sl] + prows[r1, sl] + jnp.where(mb, t1v, t0v)
                ybuf[r0, sl] = xa
                ybuf[r1, sl] = xb
                if j % 2 == 0:
                    sa0 = sa0 + xa
                    qa0 = qa0 + xa * xa
                    sb0 = sb0 + xb
                    qb0 = qb0 + xb * xb
                else:
                    sa1 = sa1 + xa
                    qa1 = qa1 + xa * xa
                    sb1 = sb1 + xb
                    qb1 = qb1 + xb * xb

            ma_v = jnp.sum(sa0 + sa1) * inv_h
            mb_v = jnp.sum(sb0 + sb1) * inv_h
            va = jnp.sum(qa0 + qa1) * inv_h - ma_v * ma_v + eps_v
            vb = jnp.sum(qb0 + qb1) * inv_h - mb_v * mb_v + eps_v
            ra = _rsqrt(va)
            rb = _rsqrt(vb)
            na = -(ma_v * ra)
            nb = -(mb_v * rb)

            for j in range(HID // L):
                sl = pl.ds(j * L, L)
                g = gam_v[sl]
                b = bet_v[sl]
                ybuf[r0, sl] = (ybuf[r0, sl] * ra + na) * g + b
                ybuf[r1, sl] = (ybuf[r1, sl] * rb + nb) * g + b
            return carry

        lax.fori_loop(0, C // 2, tok_pair, 0)

    start_gather(0, 0)

    def pair_body(i, carry):
        ck0 = 2 * i
        ck1 = ck0 + 1
        start_gather(ck1, 1)
        wait_gather(ck0, 0)

        @pl.when(i > 0)
        def _():
            out_copy(ck0 - 2, 0).wait()

        compute(ck0, 0)
        out_copy(ck0, 0).start()

        @pl.when(i < NCHUNK // 2 - 1)
        def _():
            start_gather(ck0 + 2, 0)

        wait_gather(ck1, 1)

        @pl.when(i > 0)
        def _():
            out_copy(ck1 - 2, 1).wait()

        compute(ck1, 1)
        out_copy(ck1, 1).start()
        return carry

    lax.fori_loop(0, NCHUNK // 2, pair_body, 0)
    out_copy(NCHUNK - 2, 0).wait()
    out_copy(NCHUNK - 1, 1).wait()


@jax.jit
def kernel(input_ids, token_type_ids, position_ids, word_emb, token_type_emb,
           position_emb, ln_gamma, ln_beta):
    ids = input_ids.reshape(-1).astype(jnp.int32)
    tts = token_type_ids.reshape(-1).astype(jnp.int32)
    pos = position_ids.reshape(-1).astype(jnp.int32)
    return _sc_embed(ids, tts, pos, word_emb, token_type_emb, position_emb,
                     ln_gamma, ln_beta)
